# baseline (speedup 1.0000x reference)
import jax
import jax.numpy as jnp
from jax import lax
from jax.experimental import pallas as pl
from jax.experimental.pallas import tpu as pltpu
from jax.experimental.pallas import tpu_sc as plsc

_NUM_CODEBOOKS = 32
_VOCAB = 2048
_HIDDEN = 2048
_BATCH = 1024
_TOTAL = _BATCH * _NUM_CODEBOOKS

_NC, _NS, _L = 2, 16, 16
_NW = _NC * _NS

_S_SC = 20480
_S_TC = _TOTAL - _S_SC

_PER_W = _S_SC // _NW
_K = 16
_NCHUNK = _PER_W // _K

_G = 256
_NGRP = _S_TC // _G


def _sc_body(ids_hbm, table_hbm, out_hbm, idx_v, buf0, buf1, buf2,
             g0, g1, g2, o0, o1, o2):
    c = lax.axis_index("c")
    s = lax.axis_index("s")
    wid = s * _NC + c
    base = wid * _PER_W

    pltpu.sync_copy(ids_hbm.at[pl.ds(wid * _NCHUNK, _NCHUNK)], idx_v)

    lane = lax.iota(jnp.int32, _L)
    off_even = lane * _VOCAB
    off_odd = (lane + _L) * _VOCAB

    @pl.loop(0, _NCHUNK, step=2)
    def _offsets(j0):
        idx_v[j0] = idx_v[j0] + off_even
        idx_v[j0 + 1] = idx_v[j0 + 1] + off_odd

    bufs = (buf0, buf1, buf2)
    gsem = (g0, g1, g2)
    osem = (o0, o1, o2)

    def start_gather(j, b):
        pltpu.async_copy(table_hbm.at[idx_v.at[j]], bufs[b], gsem[b])

    def start_out(j, b):
        pltpu.async_copy(bufs[b], out_hbm.at[pl.ds(base + j * _K, _K)], osem[b])

    def wait_gather(b):
        pltpu.make_async_copy(table_hbm.at[pl.ds(0, _K)], bufs[b], gsem[b]).wait()

    def wait_out(b):
        pltpu.make_async_copy(bufs[b], out_hbm.at[pl.ds(0, _K)], osem[b]).wait()

    start_gather(0, 0)
    wait_gather(0); start_out(0, 0); start_gather(1, 1)
    wait_gather(1); start_out(1, 1); start_gather(2, 2)

    @pl.loop(2, _NCHUNK - 2, step=3)
    def _pipeline(j0):
        for i in range(3):
            j = j0 + i
            b = (2 + i) % 3
            bn = (b + 1) % 3
            wait_gather(b)
            start_out(j, b)
            wait_out(bn)
            start_gather(j + 1, bn)

    jm = _NCHUNK - 2
    wait_gather(2); start_out(jm, 2); wait_out(0); start_gather(jm + 1, 0)
    wait_gather(0); start_out(jm + 1, 0); wait_out(1)
    wait_out(2)
    wait_out(0)


def _tc_body(ids_smem, table_hbm, out_hbm, buf0, buf1, g0, g1, o0, o1):
    bufs = (buf0, buf1)
    gsem = (g0, g1)
    osem = (o0, o1)

    def issue_group(g, b):
        base = g * _G
        for k in range(_G):
            idx = ids_smem[base + k] + (k % _NUM_CODEBOOKS) * _VOCAB
            pltpu.make_async_copy(
                table_hbm.at[pl.ds(idx, 1)], bufs[b].at[pl.ds(k, 1)], gsem[b]
            ).start()

    def wait_group(b):
        pltpu.make_async_copy(table_hbm.at[pl.ds(0, _G)], bufs[b], gsem[b]).wait()

    def start_write(g, b):
        pltpu.async_copy(bufs[b], out_hbm.at[pl.ds(g * _G, _G)], osem[b])

    def wait_write(b):
        pltpu.make_async_copy(bufs[b], out_hbm.at[pl.ds(0, _G)], osem[b]).wait()

    issue_group(0, 0)
    issue_group(1, 1)

    @pl.loop(0, _NGRP - 2, step=2)
    def _pipeline(g0_):
        for b in range(2):
            g = g0_ + b
            wait_group(b)
            start_write(g, b)
        for b in range(2):
            wait_write(b)
            issue_group(g0_ + b + 2, b)

    for b in range(2):
        wait_group(b)
        start_write(_NGRP - 2 + b, b)
    for b in range(2):
        wait_write(b)


@jax.jit
def kernel(input_ids, table):
    flat_ids = input_ids.astype(jnp.int32).reshape(_TOTAL)
    sc_ids = flat_ids[: _S_SC].reshape(_S_SC // _L, _L)
    tc_ids = flat_ids[_S_SC:]

    mesh = plsc.VectorSubcoreMesh(
        core_axis_name="c", subcore_axis_name="s",
        num_cores=_NC, num_subcores=_NS,
    )
    out_full = pl.kernel(
        _sc_body,
        out_type=jax.ShapeDtypeStruct((_TOTAL, _HIDDEN), jnp.float32),
        mesh=mesh,
        scratch_types=[
            pltpu.VMEM((_NCHUNK, _L), jnp.int32),
            pltpu.VMEM((_K, _HIDDEN), jnp.float32),
            pltpu.VMEM((_K, _HIDDEN), jnp.float32),
            pltpu.VMEM((_K, _HIDDEN), jnp.float32),
            pltpu.SemaphoreType.DMA,
            pltpu.SemaphoreType.DMA,
            pltpu.SemaphoreType.DMA,
            pltpu.SemaphoreType.DMA,
            pltpu.SemaphoreType.DMA,
            pltpu.SemaphoreType.DMA,
        ],
    )(sc_ids, table)

    out_tc = pl.pallas_call(
        _tc_body,
        out_shape=jax.ShapeDtypeStruct((_S_TC, _HIDDEN), jnp.float32),
        in_specs=[
            pl.BlockSpec(memory_space=pltpu.SMEM),
            pl.BlockSpec(memory_space=pltpu.HBM),
        ],
        out_specs=pl.BlockSpec(memory_space=pltpu.HBM),
        scratch_shapes=[
            pltpu.VMEM((_G, _HIDDEN), jnp.float32),
            pltpu.VMEM((_G, _HIDDEN), jnp.float32),
            pltpu.SemaphoreType.DMA,
            pltpu.SemaphoreType.DMA,
            pltpu.SemaphoreType.DMA,
            pltpu.SemaphoreType.DMA,
        ],
    )(tc_ids, table)

    out = lax.dynamic_update_slice(out_full, out_tc, (_S_SC, 0))
    return out.reshape(_BATCH, _NUM_CODEBOOKS, _HIDDEN)


# speedup vs baseline: 1.2777x; 1.2777x over previous
import jax
import jax.numpy as jnp
from jax import lax
from jax.experimental import pallas as pl
from jax.experimental.pallas import tpu as pltpu
from jax.experimental.pallas import tpu_sc as plsc

_NUM_CODEBOOKS = 32
_VOCAB = 2048
_HIDDEN = 2048
_BATCH = 1024

_NC, _NS, _L = 2, 16, 16
_NW = _NC * _NS
_TOTAL = _BATCH * _NUM_CODEBOOKS
_PER_W = _TOTAL // _NW
_K = 16
_NCHUNK = _PER_W // _K
_NBUF = 3


def _body(ids_hbm, table_hbm, out_hbm, idx_v, buf0, buf1, buf2,
          g0, g1, g2, o0, o1, o2):
    c = lax.axis_index("c")
    s = lax.axis_index("s")
    wid = s * _NC + c
    base = wid * _PER_W

    pltpu.sync_copy(ids_hbm.at[pl.ds(wid * _NCHUNK, _NCHUNK)], idx_v)

    lane = lax.iota(jnp.int32, _L)
    off_even = lane * _VOCAB
    off_odd = (lane + _L) * _VOCAB

    @pl.loop(0, _NCHUNK, step=2)
    def _offsets(j0):
        idx_v[j0] = idx_v[j0] + off_even
        idx_v[j0 + 1] = idx_v[j0 + 1] + off_odd

    bufs = (buf0, buf1, buf2)
    gsem = (g0, g1, g2)
    osem = (o0, o1, o2)

    def start_gather(j, b):
        pltpu.async_copy(table_hbm.at[idx_v.at[j]], bufs[b], gsem[b])

    def start_out(j, b):
        pltpu.async_copy(bufs[b], out_hbm.at[pl.ds(base + j * _K, _K)], osem[b])

    def wait_gather(b):
        pltpu.make_async_copy(table_hbm.at[pl.ds(0, _K)], bufs[b], gsem[b]).wait()

    def wait_out(b):
        pltpu.make_async_copy(bufs[b], out_hbm.at[pl.ds(0, _K)], osem[b]).wait()

    start_gather(0, 0)
    start_gather(1, 1)
    wait_gather(0); start_out(0, 0); start_gather(2, 2)
    wait_gather(1); start_out(1, 1); wait_out(0); start_gather(3, 0)

    @pl.loop(2, _NCHUNK - 2, step=3)
    def _pipeline(j0):
        for i in range(3):
            j = j0 + i
            b = (2 + i) % 3
            bn = (b + 2) % 3
            wait_gather(b)
            start_out(j, b)
            wait_out(bn)
            start_gather(j + 2, bn)

    jm = _NCHUNK - 2
    wait_gather(2); start_out(jm, 2); wait_out(1)
    wait_gather(0); start_out(jm + 1, 0); wait_out(2)
    wait_out(0)


@jax.jit
def kernel(input_ids, table):
    flat_ids = input_ids.astype(jnp.int32).reshape(_TOTAL // _L, _L)
    mesh = plsc.VectorSubcoreMesh(
        core_axis_name="c", subcore_axis_name="s",
        num_cores=_NC, num_subcores=_NS,
    )
    out = pl.kernel(
        _body,
        out_type=jax.ShapeDtypeStruct((_TOTAL, _HIDDEN), jnp.float32),
        mesh=mesh,
        scratch_types=[
            pltpu.VMEM((_NCHUNK, _L), jnp.int32),
            pltpu.VMEM((_K, _HIDDEN), jnp.float32),
            pltpu.VMEM((_K, _HIDDEN), jnp.float32),
            pltpu.VMEM((_K, _HIDDEN), jnp.float32),
            pltpu.SemaphoreType.DMA,
            pltpu.SemaphoreType.DMA,
            pltpu.SemaphoreType.DMA,
            pltpu.SemaphoreType.DMA,
            pltpu.SemaphoreType.DMA,
            pltpu.SemaphoreType.DMA,
        ],
    )(flat_ids, table)
    return out.reshape(_BATCH, _NUM_CODEBOOKS, _HIDDEN)


# speedup vs baseline: 1.2784x; 1.0006x over previous
import jax
import jax.numpy as jnp
from jax import lax
from jax.experimental import pallas as pl
from jax.experimental.pallas import tpu as pltpu
from jax.experimental.pallas import tpu_sc as plsc

_NUM_CODEBOOKS = 32
_VOCAB = 2048
_HIDDEN = 2048
_BATCH = 1024

_NC, _NS, _L = 2, 16, 16
_NW = _NC * _NS
_TOTAL = _BATCH * _NUM_CODEBOOKS
_PER_W = _TOTAL // _NW
_K = 16
_NCHUNK = _PER_W // _K
_NBUF = 3


def _body(ids_hbm, table_hbm, out_hbm, idx_v, buf0, buf1, buf2,
          g0, g1, g2, o0, o1, o2):
    c = lax.axis_index("c")
    s = lax.axis_index("s")
    wid = s * _NC + c
    base = wid * _PER_W

    pltpu.sync_copy(ids_hbm.at[pl.ds(wid * _NCHUNK, _NCHUNK)], idx_v)

    lane = lax.iota(jnp.int32, _L)
    off_even = lane * _VOCAB
    off_odd = (lane + _L) * _VOCAB

    @pl.loop(0, _NCHUNK, step=2)
    def _offsets(j0):
        idx_v[j0] = idx_v[j0] + off_even
        idx_v[j0 + 1] = idx_v[j0 + 1] + off_odd

    bufs = (buf0, buf1, buf2)
    gsem = (g0, g1, g2)
    osem = (o0, o1, o2)

    def start_gather(j, b):
        h = _K // 2
        pltpu.async_copy(
            table_hbm.at[idx_v.at[j, pl.ds(0, h)]], bufs[b].at[pl.ds(0, h)], gsem[b])
        pltpu.async_copy(
            table_hbm.at[idx_v.at[j, pl.ds(h, h)]], bufs[b].at[pl.ds(h, h)], gsem[b])

    def start_out(j, b):
        pltpu.async_copy(bufs[b], out_hbm.at[pl.ds(base + j * _K, _K)], osem[b])

    def wait_gather(b):
        pltpu.make_async_copy(table_hbm.at[pl.ds(0, _K)], bufs[b], gsem[b]).wait()

    def wait_out(b):
        pltpu.make_async_copy(bufs[b], out_hbm.at[pl.ds(0, _K)], osem[b]).wait()

    start_gather(0, 0)
    start_gather(1, 1)
    wait_gather(0); start_out(0, 0); start_gather(2, 2)
    wait_gather(1); start_out(1, 1); wait_out(0); start_gather(3, 0)

    @pl.loop(2, _NCHUNK - 2, step=3)
    def _pipeline(j0):
        for i in range(3):
            j = j0 + i
            b = (2 + i) % 3
            bn = (b + 2) % 3
            wait_gather(b)
            start_out(j, b)
            wait_out(bn)
            start_gather(j + 2, bn)

    jm = _NCHUNK - 2
    wait_gather(2); start_out(jm, 2); wait_out(1)
    wait_gather(0); start_out(jm + 1, 0); wait_out(2)
    wait_out(0)


@jax.jit
def kernel(input_ids, table):
    flat_ids = input_ids.astype(jnp.int32).reshape(_TOTAL // _L, _L)
    mesh = plsc.VectorSubcoreMesh(
        core_axis_name="c", subcore_axis_name="s",
        num_cores=_NC, num_subcores=_NS,
    )
    out = pl.kernel(
        _body,
        out_type=jax.ShapeDtypeStruct((_TOTAL, _HIDDEN), jnp.float32),
        mesh=mesh,
        scratch_types=[
            pltpu.VMEM((_NCHUNK, _L), jnp.int32),
            pltpu.VMEM((_K, _HIDDEN), jnp.float32),
            pltpu.VMEM((_K, _HIDDEN), jnp.float32),
            pltpu.VMEM((_K, _HIDDEN), jnp.float32),
            pltpu.SemaphoreType.DMA,
            pltpu.SemaphoreType.DMA,
            pltpu.SemaphoreType.DMA,
            pltpu.SemaphoreType.DMA,
            pltpu.SemaphoreType.DMA,
            pltpu.SemaphoreType.DMA,
        ],
    )(flat_ids, table)
    return out.reshape(_BATCH, _NUM_CODEBOOKS, _HIDDEN)


# speedup vs baseline: 1.2849x; 1.0051x over previous
import jax
import jax.numpy as jnp
from jax import lax
from jax.experimental import pallas as pl
from jax.experimental.pallas import tpu as pltpu
from jax.experimental.pallas import tpu_sc as plsc

_NUM_CODEBOOKS = 32
_VOCAB = 2048
_HIDDEN = 2048
_BATCH = 1024

_NC, _NS, _L = 2, 16, 16
_NW = _NC * _NS
_TOTAL = _BATCH * _NUM_CODEBOOKS
_PER_W = _TOTAL // _NW
_K = 16
_NCHUNK = _PER_W // _K
_NBUF = 3


def _body(ids_hbm, table_hbm, out_hbm, idx_v, buf0, buf1, buf2,
          g0, g1, g2, o0, o1, o2):
    c = lax.axis_index("c")
    s = lax.axis_index("s")
    wid = s * _NC + c
    base = wid * _PER_W

    pltpu.sync_copy(ids_hbm.at[pl.ds(wid * _NCHUNK, _NCHUNK)], idx_v)

    lane = lax.iota(jnp.int32, _L)
    off_even = lane * _VOCAB
    off_odd = (lane + _L) * _VOCAB

    bufs = (buf0, buf1, buf2)
    gsem = (g0, g1, g2)
    osem = (o0, o1, o2)

    def start_gather(j, b):
        pltpu.async_copy(table_hbm.at[idx_v.at[j]], bufs[b], gsem[b])

    def start_out(j, b):
        pltpu.async_copy(bufs[b], out_hbm.at[pl.ds(base + j * _K, _K)], osem[b])

    def wait_gather(b):
        pltpu.make_async_copy(table_hbm.at[pl.ds(0, _K)], bufs[b], gsem[b]).wait()

    def wait_out(b):
        pltpu.make_async_copy(bufs[b], out_hbm.at[pl.ds(0, _K)], osem[b]).wait()

    idx_v[0] = idx_v[0] + off_even
    idx_v[1] = idx_v[1] + off_odd
    idx_v[2] = idx_v[2] + off_even
    idx_v[3] = idx_v[3] + off_odd
    start_gather(0, 0)
    start_gather(1, 1)
    start_gather(2, 2)

    @pl.loop(4, _NCHUNK, step=2)
    def _offsets(j0):
        idx_v[j0] = idx_v[j0] + off_even
        idx_v[j0 + 1] = idx_v[j0 + 1] + off_odd

    wait_gather(0); start_out(0, 0)
    wait_gather(1); start_out(1, 1); wait_out(0); start_gather(3, 0)

    @pl.loop(2, _NCHUNK - 2, step=3)
    def _pipeline(j0):
        for i in range(3):
            j = j0 + i
            b = (2 + i) % 3
            bn = (b + 2) % 3
            wait_gather(b)
            start_out(j, b)
            wait_out(bn)
            start_gather(j + 2, bn)

    jm = _NCHUNK - 2
    wait_gather(2); start_out(jm, 2); wait_out(1)
    wait_gather(0); start_out(jm + 1, 0); wait_out(2)
    wait_out(0)


@jax.jit
def kernel(input_ids, table):
    flat_ids = input_ids.astype(jnp.int32).reshape(_TOTAL // _L, _L)
    mesh = plsc.VectorSubcoreMesh(
        core_axis_name="c", subcore_axis_name="s",
        num_cores=_NC, num_subcores=_NS,
    )
    out = pl.kernel(
        _body,
        out_type=jax.ShapeDtypeStruct((_TOTAL, _HIDDEN), jnp.float32),
        mesh=mesh,
        scratch_types=[
            pltpu.VMEM((_NCHUNK, _L), jnp.int32),
            pltpu.VMEM((_K, _HIDDEN), jnp.float32),
            pltpu.VMEM((_K, _HIDDEN), jnp.float32),
            pltpu.VMEM((_K, _HIDDEN), jnp.float32),
            pltpu.SemaphoreType.DMA,
            pltpu.SemaphoreType.DMA,
            pltpu.SemaphoreType.DMA,
            pltpu.SemaphoreType.DMA,
            pltpu.SemaphoreType.DMA,
            pltpu.SemaphoreType.DMA,
        ],
    )(flat_ids, table)
    return out.reshape(_BATCH, _NUM_CODEBOOKS, _HIDDEN)
